# Initial kernel scaffold; baseline (speedup 1.0000x reference)
import jax
import jax.numpy as jnp
from jax.experimental import pallas as pl


def kernel(x, DSM, W_a1, b_a1, W_l1, b_l1, W_l2, b_l2, W_a2, b_a2, W_a3, b_a3, W_ah, b_ah, W_vh, b_vh):
    raise NotImplementedError("write your pallas kernel here")


# speedup vs baseline: 4162.6537x; 4162.6537x over previous
import jax
import jax.numpy as jnp
from jax.experimental import pallas as pl
from jax.experimental.pallas import tpu as pltpu


def _policy_body(s_ref, x_ref, dsm_ref, ap_ref, sv_ref):
    dsm = dsm_ref[...]
    x = x_ref[...]
    n = dsm.shape[0]

    w_a1 = s_ref[0]
    b_a1 = s_ref[1]
    w_l1 = s_ref[2]
    b_l1 = s_ref[3]
    w_l2 = s_ref[4]
    b_l2 = s_ref[5]
    w_a2 = s_ref[6]
    b_a2 = s_ref[7]
    w_a3 = s_ref[8]
    b_a3 = s_ref[9]
    w_vh = s_ref[10]
    b_vh = s_ref[11]

    deg = jnp.sum(dsm, axis=0) + 1.0
    dinv = jnp.where(deg > 0, jax.lax.rsqrt(deg), 0.0)

    y = w_a1 * x
    z = y * dinv[None, :]
    t = jnp.dot(z, dsm, preferred_element_type=jnp.float32)
    h = jnp.maximum(t * dinv[None, :] + y * (dinv * dinv)[None, :] + b_a1, 0.0)

    h = jnp.maximum(w_l1 * h + b_l1, 0.0)
    h = jnp.maximum(w_l2 * h + b_l2, 0.0)

    inv_np1 = 1.0 / (n + 1.0)
    y = w_a2 * h
    h = jnp.maximum((jnp.sum(y, axis=1, keepdims=True) + y) * inv_np1 + b_a2, 0.0)
    y = w_a3 * h
    h = jnp.maximum((jnp.sum(y, axis=1, keepdims=True) + y) * inv_np1 + b_a3, 0.0)

    ap_ref[...] = jnp.ones_like(x)

    m = jnp.max(h)
    sv_ref[...] = jnp.full((1, 1), w_vh * m + b_vh, jnp.float32)


def kernel(x, DSM, W_a1, b_a1, W_l1, b_l1, W_l2, b_l2, W_a2, b_a2,
           W_a3, b_a3, W_ah, b_ah, W_vh, b_vh):
    B, N = x.shape
    scal = jnp.stack([
        W_a1[0, 0], b_a1[0], W_l1[0, 0], b_l1[0], W_l2[0, 0], b_l2[0],
        W_a2[0, 0], b_a2[0], W_a3[0, 0], b_a3[0], W_vh[0, 0], b_vh[0],
    ]).astype(jnp.float32)

    ap, sv = pl.pallas_call(
        _policy_body,
        in_specs=[
            pl.BlockSpec(memory_space=pltpu.SMEM),
            pl.BlockSpec(memory_space=pltpu.VMEM),
            pl.BlockSpec(memory_space=pltpu.VMEM),
        ],
        out_specs=[
            pl.BlockSpec(memory_space=pltpu.VMEM),
            pl.BlockSpec(memory_space=pltpu.VMEM),
        ],
        out_shape=[
            jax.ShapeDtypeStruct((B, N), jnp.float32),
            jax.ShapeDtypeStruct((1, 1), jnp.float32),
        ],
    )(scal, x, DSM)

    return ap[:, :, None], sv[:, :, None]
